# baseline (speedup 1.0000x reference)
import functools

import jax
import jax.numpy as jnp
from jax import lax
from jax.experimental import pallas as pl
from jax.experimental.pallas import tpu as pltpu
from jax.experimental.pallas import tpu_sc as plsc

BATCH = 16384
D = 32
LANES = 16
NUM_CORES = 2
NUM_SUBCORES = 16
NUM_WORKERS = NUM_CORES * NUM_SUBCORES
B_PER_W = BATCH // NUM_WORKERS
IDX_CHUNK = 128
N_CHUNKS = B_PER_W // IDX_CHUNK


def _vad_body(ids_hbm, eps_hbm, mu_hbm, lv_hbm,
              lat_out, mu_out, lv_out,
              idx_v, mu_v, lv_v, eps_v, lat_v, gsem, esem, osem):
    wid = lax.axis_index("s") * NUM_CORES + lax.axis_index("c")
    base = wid * B_PER_W

    pltpu.sync_copy(ids_hbm.at[wid], idx_v)

    gathers = []
    for j in range(N_CHUNKS):
        dst = pl.ds(j * IDX_CHUNK, IDX_CHUNK)
        gathers.append(pltpu.async_copy(mu_hbm.at[idx_v.at[j]], mu_v.at[dst], gsem))
        gathers.append(pltpu.async_copy(lv_hbm.at[idx_v.at[j]], lv_v.at[dst], gsem))
    eps_cp = pltpu.async_copy(eps_hbm.at[pl.ds(base, B_PER_W)], eps_v, esem)
    for g in gathers:
        g.wait()

    mu_cp = pltpu.async_copy(mu_v, mu_out.at[pl.ds(base, B_PER_W)], osem)
    lv_cp = pltpu.async_copy(lv_v, lv_out.at[pl.ds(base, B_PER_W)], osem)
    eps_cp.wait()

    def body(i, carry):
        for j in range(D // LANES):
            sl = pl.ds(j * LANES, LANES)
            m = mu_v[i, sl]
            v = lv_v[i, sl]
            e = eps_v[i, sl]
            lat_v[i, sl] = m + e * jnp.exp(v * 0.5)
        return carry

    lax.fori_loop(0, B_PER_W, body, 0)

    pltpu.sync_copy(lat_v, lat_out.at[pl.ds(base, B_PER_W)])
    mu_cp.wait()
    lv_cp.wait()


@functools.partial(
    pl.kernel,
    out_type=(
        jax.ShapeDtypeStruct((BATCH, D), jnp.float32),
        jax.ShapeDtypeStruct((BATCH, D), jnp.float32),
        jax.ShapeDtypeStruct((BATCH, D), jnp.float32),
    ),
    mesh=plsc.VectorSubcoreMesh(core_axis_name="c", subcore_axis_name="s"),
    compiler_params=pltpu.CompilerParams(use_tc_tiling_on_sc=False),
    scratch_types=[
        pltpu.VMEM((N_CHUNKS, IDX_CHUNK), jnp.int32),
        pltpu.VMEM((B_PER_W, D), jnp.float32),
        pltpu.VMEM((B_PER_W, D), jnp.float32),
        pltpu.VMEM((B_PER_W, D), jnp.float32),
        pltpu.VMEM((B_PER_W, D), jnp.float32),
        pltpu.SemaphoreType.DMA,
        pltpu.SemaphoreType.DMA,
        pltpu.SemaphoreType.DMA,
    ],
)
def _vad_kernel(ids_hbm, eps_hbm, mu_hbm, lv_hbm, lat_out, mu_out, lv_out,
                idx_v, mu_v, lv_v, eps_v, lat_v, gsem, esem, osem):
    _vad_body(ids_hbm, eps_hbm, mu_hbm, lv_hbm, lat_out, mu_out, lv_out,
              idx_v, mu_v, lv_v, eps_v, lat_v, gsem, esem, osem)


@jax.jit
def kernel(instance_ids, eps, weight_mu, weight_logvar):
    ids3 = instance_ids.reshape(NUM_WORKERS, N_CHUNKS, IDX_CHUNK)
    lat, mu, lv = _vad_kernel(ids3, eps, weight_mu, weight_logvar)
    return (lat, mu, lv)
